# baseline (speedup 1.0000x reference)
import jax
import jax.numpy as jnp
from jax import lax
from jax.experimental import pallas as pl
from jax.experimental.pallas import tpu as pltpu

_TOTAL = 32768
_D = 128
_NSEG = 16
_BLK = 4096
_NBLK = _TOTAL // _BLK


def _body(bounds_ref, flat_ref, w_ref, out_ref, acc_ref, mw_ref, cache_ref):
    p = pl.program_id(0)
    b = pl.program_id(1)

    base = b * _BLK
    rows = bounds_ref[0:_NSEG, :] + base
    starts = bounds_ref[_NSEG:2 * _NSEG, :]
    ends = bounds_ref[2 * _NSEG:3 * _NSEG, :]
    onehot_t = ((rows >= starts) & (rows < ends)).astype(jnp.float32)

    @pl.when((p == 0) & (b == 0))
    def _init():
        acc_ref[...] = jnp.zeros_like(acc_ref)

    @pl.when(p == 0)
    def _phase0():
        blk = flat_ref[...]
        cache_ref[pl.ds(base, _BLK), :] = jnp.dot(
            blk, w_ref[...], preferred_element_type=jnp.float32)
        acc_ref[...] += lax.dot_general(
            onehot_t, blk, (((1,), (0,)), ((), ())),
            preferred_element_type=jnp.float32)

    @pl.when((p == 1) & (b == 0))
    def _means():
        counts = (bounds_ref[2 * _NSEG:3 * _NSEG, 0:_D]
                  - bounds_ref[_NSEG:2 * _NSEG, 0:_D]).astype(jnp.float32)
        mean = acc_ref[...] / jnp.maximum(counts, 1.0)
        mw_ref[...] = jnp.dot(mean, w_ref[...],
                              preferred_element_type=jnp.float32)

    @pl.when(p == 1)
    def _phase1():
        corr = lax.dot_general(
            onehot_t, mw_ref[...], (((0,), (0,)), ((), ())),
            preferred_element_type=jnp.float32)
        out_ref[...] = cache_ref[pl.ds(base, _BLK), :] - corr


def kernel(flat, cu_seqlens, W):
    rows_base = jax.lax.broadcasted_iota(jnp.int32, (_NSEG, _BLK), 1)
    starts_b = jnp.broadcast_to(cu_seqlens[:_NSEG, None], (_NSEG, _BLK))
    ends_b = jnp.broadcast_to(cu_seqlens[1:_NSEG + 1, None], (_NSEG, _BLK))
    bounds = jnp.concatenate([rows_base, starts_b, ends_b], axis=0)
    return pl.pallas_call(
        _body,
        grid=(2, _NBLK),
        in_specs=[
            pl.BlockSpec((3 * _NSEG, _BLK), lambda p, b: (0, 0)),
            pl.BlockSpec((_BLK, _D),
                         lambda p, b: (b * (1 - p) + (_NBLK - 1) * p, 0)),
            pl.BlockSpec((_D, _D), lambda p, b: (0, 0)),
        ],
        out_specs=pl.BlockSpec((_BLK, _D), lambda p, b: (b * p, 0)),
        out_shape=jax.ShapeDtypeStruct((_TOTAL, _D), jnp.float32),
        scratch_shapes=[
            pltpu.VMEM((_NSEG, _D), jnp.float32),
            pltpu.VMEM((_NSEG, _D), jnp.float32),
            pltpu.VMEM((_TOTAL, _D), jnp.float32),
        ],
        compiler_params=pltpu.CompilerParams(
            dimension_semantics=("arbitrary", "arbitrary"),
        ),
    )(bounds, flat, W)


# speedup vs baseline: 1.0623x; 1.0623x over previous
import jax
import jax.numpy as jnp
from jax import lax
from jax.experimental import pallas as pl
from jax.experimental.pallas import tpu as pltpu

_TOTAL = 32768
_D = 128
_NSEG = 16
_BLK = 16384
_NBLK = _TOTAL // _BLK


def _body(bounds_ref, flat_ref, w_ref, out_ref, acc_ref, mw_ref, cache_ref):
    p = pl.program_id(0)
    b = pl.program_id(1)

    base = b * _BLK
    rows = bounds_ref[0:_NSEG, :] + base
    starts = bounds_ref[_NSEG:2 * _NSEG, :]
    ends = bounds_ref[2 * _NSEG:3 * _NSEG, :]
    onehot_t = ((rows >= starts) & (rows < ends)).astype(jnp.float32)

    @pl.when((p == 0) & (b == 0))
    def _init():
        acc_ref[...] = jnp.zeros_like(acc_ref)

    @pl.when(p == 0)
    def _phase0():
        blk = flat_ref[...]
        cache_ref[pl.ds(base, _BLK), :] = jnp.dot(
            blk, w_ref[...], preferred_element_type=jnp.float32)
        acc_ref[...] += lax.dot_general(
            onehot_t, blk, (((1,), (0,)), ((), ())),
            preferred_element_type=jnp.float32)

    @pl.when((p == 1) & (b == 0))
    def _means():
        counts = (bounds_ref[2 * _NSEG:3 * _NSEG, 0:_D]
                  - bounds_ref[_NSEG:2 * _NSEG, 0:_D]).astype(jnp.float32)
        mean = acc_ref[...] / jnp.maximum(counts, 1.0)
        mw_ref[...] = jnp.dot(mean, w_ref[...],
                              preferred_element_type=jnp.float32)

    @pl.when(p == 1)
    def _phase1():
        corr = lax.dot_general(
            onehot_t, mw_ref[...], (((0,), (0,)), ((), ())),
            preferred_element_type=jnp.float32)
        out_ref[...] = cache_ref[pl.ds(base, _BLK), :] - corr


def kernel(flat, cu_seqlens, W):
    rows_base = jax.lax.broadcasted_iota(jnp.int32, (_NSEG, _BLK), 1)
    starts_b = jnp.broadcast_to(cu_seqlens[:_NSEG, None], (_NSEG, _BLK))
    ends_b = jnp.broadcast_to(cu_seqlens[1:_NSEG + 1, None], (_NSEG, _BLK))
    bounds = jnp.concatenate([rows_base, starts_b, ends_b], axis=0)
    return pl.pallas_call(
        _body,
        grid=(2, _NBLK),
        in_specs=[
            pl.BlockSpec((3 * _NSEG, _BLK), lambda p, b: (0, 0)),
            pl.BlockSpec((_BLK, _D),
                         lambda p, b: (b * (1 - p) + (_NBLK - 1) * p, 0)),
            pl.BlockSpec((_D, _D), lambda p, b: (0, 0)),
        ],
        out_specs=pl.BlockSpec((_BLK, _D), lambda p, b: (b * p, 0)),
        out_shape=jax.ShapeDtypeStruct((_TOTAL, _D), jnp.float32),
        scratch_shapes=[
            pltpu.VMEM((_NSEG, _D), jnp.float32),
            pltpu.VMEM((_NSEG, _D), jnp.float32),
            pltpu.VMEM((_TOTAL, _D), jnp.float32),
        ],
        compiler_params=pltpu.CompilerParams(
            dimension_semantics=("arbitrary", "arbitrary"),
        ),
    )(bounds, flat, W)


# speedup vs baseline: 1.2274x; 1.1554x over previous
import jax
import jax.numpy as jnp
from jax import lax
from jax.experimental import pallas as pl
from jax.experimental.pallas import tpu as pltpu

_TOTAL = 32768
_D = 128
_NSEG = 16
_CH = 4096
_NCH = _TOTAL // _CH


def _body(bounds_ref, flat_ref, w_ref, out_ref, vin_ref, vout_ref, acc_ref,
          insem, outsem):
    def in_copy(i):
        return pltpu.make_async_copy(
            flat_ref.at[pl.ds(i * _CH, _CH), :],
            vin_ref.at[pl.ds(i * _CH, _CH), :],
            insem.at[i])

    def out_copy(i):
        return pltpu.make_async_copy(
            vout_ref.at[pl.ds(i * _CH, _CH), :],
            out_ref.at[pl.ds(i * _CH, _CH), :],
            outsem.at[i])

    for i in range(_NCH):
        in_copy(i).start()

    starts = bounds_ref[_NSEG:2 * _NSEG, :]
    ends = bounds_ref[2 * _NSEG:3 * _NSEG, :]

    def onehot(i):
        rows = bounds_ref[0:_NSEG, :] + i * _CH
        return ((rows >= starts) & (rows < ends)).astype(jnp.float32)

    acc_ref[...] = jnp.zeros_like(acc_ref)
    for i in range(_NCH):
        in_copy(i).wait()
        blk = vin_ref[pl.ds(i * _CH, _CH), :]
        vout_ref[pl.ds(i * _CH, _CH), :] = jnp.dot(
            blk, w_ref[...], preferred_element_type=jnp.float32)
        acc_ref[...] += lax.dot_general(
            onehot(i), blk, (((1,), (0,)), ((), ())),
            preferred_element_type=jnp.float32)

    counts = (bounds_ref[2 * _NSEG:3 * _NSEG, 0:_D]
              - bounds_ref[_NSEG:2 * _NSEG, 0:_D]).astype(jnp.float32)
    mean = acc_ref[...] / jnp.maximum(counts, 1.0)
    mw = jnp.dot(mean, w_ref[...], preferred_element_type=jnp.float32)

    for i in range(_NCH):
        corr = lax.dot_general(
            onehot(i), mw, (((0,), (0,)), ((), ())),
            preferred_element_type=jnp.float32)
        vout_ref[pl.ds(i * _CH, _CH), :] = (
            vout_ref[pl.ds(i * _CH, _CH), :] - corr)
        out_copy(i).start()

    for i in range(_NCH):
        out_copy(i).wait()


def kernel(flat, cu_seqlens, W):
    rows_base = jax.lax.broadcasted_iota(jnp.int32, (_NSEG, _CH), 1)
    starts_b = jnp.broadcast_to(cu_seqlens[:_NSEG, None], (_NSEG, _CH))
    ends_b = jnp.broadcast_to(cu_seqlens[1:_NSEG + 1, None], (_NSEG, _CH))
    bounds = jnp.concatenate([rows_base, starts_b, ends_b], axis=0)
    return pl.pallas_call(
        _body,
        in_specs=[
            pl.BlockSpec(memory_space=pltpu.VMEM),
            pl.BlockSpec(memory_space=pl.ANY),
            pl.BlockSpec(memory_space=pltpu.VMEM),
        ],
        out_specs=pl.BlockSpec(memory_space=pl.ANY),
        out_shape=jax.ShapeDtypeStruct((_TOTAL, _D), jnp.float32),
        scratch_shapes=[
            pltpu.VMEM((_TOTAL, _D), jnp.float32),
            pltpu.VMEM((_TOTAL, _D), jnp.float32),
            pltpu.VMEM((_NSEG, _D), jnp.float32),
            pltpu.SemaphoreType.DMA((_NCH,)),
            pltpu.SemaphoreType.DMA((_NCH,)),
        ],
    )(bounds, flat, W)
